# baseline (speedup 1.0000x reference)
import functools

import jax
import jax.numpy as jnp
from jax import lax
from jax.experimental import pallas as pl
from jax.experimental.pallas import tpu as pltpu
from jax.experimental.pallas import tpu_sc as plsc


def _sc_gather(tok_table, idx_flat):
    info = plsc.get_sparse_core_info()
    nw = info.num_cores * info.num_subcores
    n = idx_flat.shape[0]
    d = tok_table.shape[1]
    b_per_w = n // nw
    mesh = plsc.VectorSubcoreMesh(core_axis_name="c", subcore_axis_name="s")

    @functools.partial(
        pl.kernel,
        mesh=mesh,
        out_type=jax.ShapeDtypeStruct((n, d), jnp.float32),
        scratch_types=[
            pltpu.VMEM((b_per_w,), jnp.int32),
            pltpu.VMEM((b_per_w, d), jnp.float32),
            pltpu.SemaphoreType.DMA,
        ],
        compiler_params=pltpu.CompilerParams(use_tc_tiling_on_sc=True),
    )
    def gather_k(table_hbm, idx_hbm, out_hbm, idx_v, rows_v, sem):
        nc = info.num_cores
        wid = lax.axis_index("s") * nc + lax.axis_index("c")
        base = wid * b_per_w
        pltpu.sync_copy(idx_hbm.at[pl.ds(base, b_per_w)], idx_v)
        vecs = [idx_v[pl.ds(c * 16, 16)] for c in range(b_per_w // 16)]
        copies = [
            pltpu.async_copy(
                table_hbm.at[pl.ds(vecs[i // 16][i % 16], 1)],
                rows_v.at[pl.ds(i, 1)],
                sem,
            )
            for i in range(b_per_w)
        ]
        for c in copies:
            c.wait()
        pltpu.sync_copy(rows_v, out_hbm.at[pl.ds(base, b_per_w)])

    return gather_k(tok_table, idx_flat)


_VT = 2048


def _head_body(x_ref, pos_ref, w_ref, b_ref, o_ref):
    x = x_ref[...] + pos_ref[...]
    o_ref[...] = (
        jnp.dot(x, w_ref[...], preferred_element_type=jnp.float32) + b_ref[...]
    )


def _head(x_tok, pos_full, W, b2):
    m, d = x_tok.shape
    v = W.shape[1]
    nv = pl.cdiv(v, _VT)
    return pl.pallas_call(
        _head_body,
        grid=(nv,),
        in_specs=[
            pl.BlockSpec((m, d), lambda j: (0, 0)),
            pl.BlockSpec((m, d), lambda j: (0, 0)),
            pl.BlockSpec((d, _VT), lambda j: (0, j)),
            pl.BlockSpec((1, _VT), lambda j: (0, j)),
        ],
        out_specs=pl.BlockSpec((m, _VT), lambda j: (0, j)),
        out_shape=jax.ShapeDtypeStruct((m, v), jnp.float32),
        compiler_params=pltpu.CompilerParams(
            dimension_semantics=("arbitrary",),
        ),
    )(x_tok, pos_full, W, b2)


def kernel(idx, tok_table, pos_table, W, b):
    bb, t = idx.shape
    d = tok_table.shape[1]
    idx_flat = idx.reshape(-1).astype(jnp.int32)
    x_tok = _sc_gather(tok_table, idx_flat)
    pos_full = jnp.broadcast_to(pos_table[None], (bb, t, d)).reshape(bb * t, d)
    out = _head(x_tok, pos_full, W, b.reshape(1, -1))
    return out.reshape(bb, t, -1)


# speedup vs baseline: 1.9318x; 1.9318x over previous
import functools

import jax
import jax.numpy as jnp
from jax import lax
from jax.experimental import pallas as pl
from jax.experimental.pallas import tpu as pltpu
from jax.experimental.pallas import tpu_sc as plsc


_CH = 8


def _sc_gather(table_t, idx_flat):
    info = plsc.get_sparse_core_info()
    nw = info.num_cores * info.num_subcores
    emb = table_t.shape[0]
    n = idx_flat.shape[0]
    c_per_w = n // nw
    mesh = plsc.VectorSubcoreMesh(core_axis_name="c", subcore_axis_name="s")

    @functools.partial(
        pl.kernel,
        mesh=mesh,
        out_type=jax.ShapeDtypeStruct((n, emb), jnp.float32),
        scratch_types=[
            pltpu.VMEM((c_per_w,), jnp.int32),
            pltpu.VMEM((_CH, emb, 128), jnp.float32),
            pltpu.VMEM((c_per_w, emb), jnp.float32),
            pltpu.SemaphoreType.DMA,
        ],
        compiler_params=pltpu.CompilerParams(
            use_tc_tiling_on_sc=True, needs_layout_passes=False
        ),
    )
    def gather_k(tab_hbm, idx_hbm, out_hbm, idx_v, win_v, x_v, sem):
        wid = lax.axis_index("s") * info.num_cores + lax.axis_index("c")
        base = wid * c_per_w
        pltpu.sync_copy(idx_hbm.at[pl.ds(base, c_per_w)], idx_v)
        iv = [idx_v[pl.ds(c * 16, 16)] for c in range(c_per_w // 16)]
        dvecs = [lax.iota(jnp.int32, 16) + 16 * j for j in range(emb // 16)]
        for ch in range(c_per_w // _CH):
            copies = []
            for k2 in range(_CH):
                i = ch * _CH + k2
                s = iv[i // 16][i % 16]
                w0 = pl.multiple_of((s // 128) * 128, 128)
                copies.append(
                    pltpu.async_copy(
                        tab_hbm.at[:, pl.ds(w0, 128)], win_v.at[k2], sem
                    )
                )
            for c in copies:
                c.wait()
            for k2 in range(_CH):
                i = ch * _CH + k2
                s = iv[i // 16][i % 16]
                k_splat = jnp.full((16,), k2, jnp.int32)
                l_splat = jnp.full((16,), s % 128, jnp.int32)
                for j in range(emb // 16):
                    x_v[i, pl.ds(16 * j, 16)] = plsc.load_gather(
                        win_v, [k_splat, dvecs[j], l_splat]
                    )
        pltpu.sync_copy(x_v, out_hbm.at[pl.ds(base, c_per_w)])

    return gather_k(table_t, idx_flat)


_VT = 4096


def _head_body(x_ref, pos_ref, w_ref, b_ref, o_ref, xs_ref):
    bb, _, t = xs_ref.shape
    j = pl.program_id(0)
    bq = pl.program_id(1)

    @pl.when((j == 0) & (bq == 0))
    def _():
        for b0 in range(bb):
            xs_ref[b0] = jnp.transpose(
                x_ref[pl.ds(b0 * t, t), :] + pos_ref[pl.ds(b0 * t, t), :]
            )

    y = lax.dot_general(
        w_ref[...],
        xs_ref[bq],
        (((0,), (0,)), ((), ())),
        preferred_element_type=jnp.float32,
    )
    o_ref[...] = (y + jnp.transpose(b_ref[...])).reshape(1, -1, t)


def _head_t(x, pos_full, W, b2, bb, t):
    m, d = x.shape
    v = W.shape[1]
    nv = pl.cdiv(v, _VT)
    return pl.pallas_call(
        _head_body,
        grid=(nv, bb),
        in_specs=[
            pl.BlockSpec((m, d), lambda j, bq: (0, 0)),
            pl.BlockSpec((m, d), lambda j, bq: (0, 0)),
            pl.BlockSpec((d, _VT), lambda j, bq: (0, j)),
            pl.BlockSpec((1, _VT), lambda j, bq: (0, j)),
        ],
        out_specs=pl.BlockSpec((1, _VT, t), lambda j, bq: (bq, j, 0)),
        out_shape=jax.ShapeDtypeStruct((bb, v, t), jnp.float32),
        scratch_shapes=[pltpu.VMEM((bb, d, t), jnp.float32)],
        compiler_params=pltpu.CompilerParams(
            dimension_semantics=("arbitrary", "arbitrary"),
        ),
    )(x, pos_full, W, b2)


def kernel(idx, tok_table, pos_table, W, b):
    bb, t = idx.shape
    d = tok_table.shape[1]
    idx_flat = idx.reshape(-1).astype(jnp.int32)
    x = _sc_gather(tok_table.T, idx_flat)
    pos_full = jnp.broadcast_to(pos_table[None], (bb, t, d)).reshape(bb * t, d)
    out_t = _head_t(x, pos_full, W, b.reshape(1, -1), bb, t)
    return jnp.swapaxes(out_t, 1, 2)


# speedup vs baseline: 2.7056x; 1.4005x over previous
import functools

import jax
import jax.numpy as jnp
from jax import lax
from jax.experimental import pallas as pl
from jax.experimental.pallas import tpu as pltpu
from jax.experimental.pallas import tpu_sc as plsc


_CH = 8


def _sc_gather(table_t, idx_flat):
    info = plsc.get_sparse_core_info()
    nw = info.num_cores * info.num_subcores
    emb = table_t.shape[0]
    n = idx_flat.shape[0]
    c_per_w = n // nw
    mesh = plsc.VectorSubcoreMesh(core_axis_name="c", subcore_axis_name="s")

    @functools.partial(
        pl.kernel,
        mesh=mesh,
        out_type=jax.ShapeDtypeStruct((n, emb), jnp.float32),
        scratch_types=[
            pltpu.VMEM((c_per_w,), jnp.int32),
            pltpu.VMEM((_CH, emb, 128), jnp.float32),
            pltpu.VMEM((c_per_w, emb), jnp.float32),
            pltpu.SemaphoreType.DMA,
        ],
        compiler_params=pltpu.CompilerParams(
            use_tc_tiling_on_sc=True, needs_layout_passes=False
        ),
    )
    def gather_k(tab_hbm, idx_hbm, out_hbm, idx_v, win_v, x_v, sem):
        wid = lax.axis_index("s") * info.num_cores + lax.axis_index("c")
        base = wid * c_per_w
        pltpu.sync_copy(idx_hbm.at[pl.ds(base, c_per_w)], idx_v)
        iv = [idx_v[pl.ds(c * 16, 16)] for c in range(c_per_w // 16)]
        dvecs = [lax.iota(jnp.int32, 16) + 16 * j for j in range(emb // 16)]
        for ch in range(c_per_w // _CH):
            copies = []
            for k2 in range(_CH):
                i = ch * _CH + k2
                s = iv[i // 16][i % 16]
                w0 = pl.multiple_of((s // 128) * 128, 128)
                copies.append(
                    pltpu.async_copy(
                        tab_hbm.at[:, pl.ds(w0, 128)], win_v.at[k2], sem
                    )
                )
            for c in copies:
                c.wait()
            for k2 in range(_CH):
                i = ch * _CH + k2
                s = iv[i // 16][i % 16]
                k_splat = jnp.full((16,), k2, jnp.int32)
                l_splat = jnp.full((16,), s % 128, jnp.int32)
                for j in range(emb // 16):
                    x_v[i, pl.ds(16 * j, 16)] = plsc.load_gather(
                        win_v, [k_splat, dvecs[j], l_splat]
                    )
        pltpu.sync_copy(x_v, out_hbm.at[pl.ds(base, c_per_w)])

    return gather_k(table_t, idx_flat)


_VT = 4096


def _head_body(x_ref, pos_ref, w_ref, b_ref, o_ref, xs_ref):
    bb, _, t = xs_ref.shape
    j = pl.program_id(0)

    @pl.when(j == 0)
    def _():
        for b0 in range(bb):
            xs_ref[b0] = jnp.transpose(
                x_ref[pl.ds(b0 * t, t), :] + pos_ref[pl.ds(b0 * t, t), :]
            )

    bc = jnp.transpose(b_ref[...])
    for b0 in range(bb):
        y = lax.dot_general(
            w_ref[...],
            xs_ref[b0],
            (((0,), (0,)), ((), ())),
            preferred_element_type=jnp.float32,
        )
        o_ref[b0] = y + bc


def _head_t(x, pos_full, W, b2, bb, t):
    m, d = x.shape
    v = W.shape[1]
    nv = pl.cdiv(v, _VT)
    return pl.pallas_call(
        _head_body,
        grid=(nv,),
        in_specs=[
            pl.BlockSpec((m, d), lambda j: (0, 0)),
            pl.BlockSpec((m, d), lambda j: (0, 0)),
            pl.BlockSpec((d, _VT), lambda j: (0, j)),
            pl.BlockSpec((1, _VT), lambda j: (0, j)),
        ],
        out_specs=pl.BlockSpec((bb, _VT, t), lambda j: (0, j, 0)),
        out_shape=jax.ShapeDtypeStruct((bb, v, t), jnp.float32),
        scratch_shapes=[
            pltpu.VMEM((bb, d, t), jnp.float32),
        ],
        compiler_params=pltpu.CompilerParams(
            dimension_semantics=("arbitrary",),
            fuse_transposed_lhs_in_matmul=True,
        ),
    )(x, pos_full, W, b2)


def kernel(idx, tok_table, pos_table, W, b):
    bb, t = idx.shape
    d = tok_table.shape[1]
    idx_flat = idx.reshape(-1).astype(jnp.int32)
    x = _sc_gather(tok_table.T, idx_flat)
    pos_full = jnp.broadcast_to(pos_table[None], (bb, t, d)).reshape(bb * t, d)
    out_t = _head_t(x, pos_full, W, b.reshape(1, -1), bb, t)
    return jnp.swapaxes(out_t, 1, 2)
